# baseline (speedup 1.0000x reference)
import functools

import jax
import jax.numpy as jnp
from jax import lax
from jax.experimental import pallas as pl
from jax.experimental.pallas import tpu as pltpu
from jax.experimental.pallas import tpu_sc as plsc

_VOCAB = 1000000
_D = 64
_B = 16384
_W = 16
_TOT = _B * _W
_NC = 2
_NS = 16
_NW = _NC * _NS
_PER_W = _TOT // _NW
_CHUNK = 128
_NCH = _PER_W // _CHUNK

_mesh = plsc.VectorSubcoreMesh(core_axis_name="c", subcore_axis_name="s")


@functools.partial(
    pl.kernel,
    mesh=_mesh,
    out_type=jax.ShapeDtypeStruct((_TOT, _D), jnp.float32),
    compiler_params=pltpu.CompilerParams(use_tc_tiling_on_sc=False),
    scratch_types=[
        pltpu.VMEM((_NCH, _CHUNK), jnp.int32),
        pltpu.VMEM((_CHUNK, _D), jnp.float32),
        pltpu.SemaphoreType.DMA,
    ],
)
def _gather_kernel(idx_hbm, table_hbm, out_hbm, idx_v, rows_v, gsem):
    wid = lax.axis_index("s") * _NC + lax.axis_index("c")
    base = wid * _PER_W
    pltpu.sync_copy(idx_hbm.at[wid], idx_v)

    def body(c, carry):
        pltpu.async_copy(table_hbm.at[idx_v.at[c]], rows_v, gsem).wait()
        pltpu.sync_copy(rows_v, out_hbm.at[pl.ds(base + c * _CHUNK, _CHUNK)])
        return carry

    lax.fori_loop(0, _NCH, body, 0)


def kernel(sentence, word_embd):
    idx = sentence.astype(jnp.int32).reshape(_NW, _NCH, _CHUNK)
    out = _gather_kernel(idx, word_embd)
    return out.reshape(_B, _W, _D)


# speedup vs baseline: 1.0577x; 1.0577x over previous
import functools

import jax
import jax.numpy as jnp
from jax import lax
from jax.experimental import pallas as pl
from jax.experimental.pallas import tpu as pltpu
from jax.experimental.pallas import tpu_sc as plsc

_VOCAB = 1000000
_D = 64
_B = 16384
_W = 16
_TOT = _B * _W
_NC = 2
_NS = 16
_NW = _NC * _NS
_PER_W = _TOT // _NW
_CHUNK = 128
_NCH = _PER_W // _CHUNK
_NBUF = 4

_mesh = plsc.VectorSubcoreMesh(core_axis_name="c", subcore_axis_name="s")


@functools.partial(
    pl.kernel,
    mesh=_mesh,
    out_type=jax.ShapeDtypeStruct((_TOT, _D), jnp.float32),
    compiler_params=pltpu.CompilerParams(use_tc_tiling_on_sc=False),
    scratch_types=[
        pltpu.VMEM((_NCH, _CHUNK), jnp.int32),
        pltpu.VMEM((_NBUF, _CHUNK, _D), jnp.float32),
        [pltpu.SemaphoreType.DMA] * _NBUF,
        [pltpu.SemaphoreType.DMA] * _NBUF,
    ],
)
def _gather_kernel(idx_hbm, table_hbm, out_hbm, idx_v, rows_v, gsems, ssems):
    wid = lax.axis_index("s") * _NC + lax.axis_index("c")
    base = wid * _PER_W
    pltpu.sync_copy(idx_hbm.at[wid], idx_v)

    def gather(c, b):
        return pltpu.make_async_copy(
            table_hbm.at[idx_v.at[c]], rows_v.at[b], gsems[b])

    def store(c, b):
        return pltpu.make_async_copy(
            rows_v.at[b], out_hbm.at[pl.ds(base + c * _CHUNK, _CHUNK)],
            ssems[b])

    for b in range(_NBUF - 1):
        gather(b, b).start()

    def group(g, carry):
        c0 = g * _NBUF
        for b in range(_NBUF):
            c = c0 + b
            nc = c + _NBUF - 1
            nb = (b + _NBUF - 1) % _NBUF

            @pl.when(nc < _NCH)
            def _():
                @pl.when(c >= 1)
                def _():
                    store(c - 1, nb).wait()
                gather(nc, nb).start()

            gather(c, b).wait()
            store(c, b).start()
        return carry

    lax.fori_loop(0, _NCH // _NBUF, group, 0)
    for b in range(_NBUF):
        store(0, b).wait()


def kernel(sentence, word_embd):
    idx = sentence.astype(jnp.int32).reshape(_NW, _NCH, _CHUNK)
    out = _gather_kernel(idx, word_embd)
    return out.reshape(_B, _W, _D)
